# baseline (speedup 1.0000x reference)
import functools

import jax
import jax.numpy as jnp
from jax import lax
from jax.experimental import pallas as pl
from jax.experimental.pallas import tpu as pltpu
from jax.experimental.pallas import tpu_sc as plsc

BATCH = 16384
EMB = 64
ROWW = 128
NC = 2
NS = 16
NW = NC * NS
BPW = BATCH // NW
HALF = BPW // 2
CHUNK = 128
NCHUNK = HALF // CHUNK
LANES = 16


def _sc_body(users_hbm, pos_hbm, neg_hbm, item2_hbm, table2_hbm,
             pos_out_hbm, neg_out_hbm,
             raw, gidx, urows, prows, nrows, outp, outn, sem):
    wid = lax.axis_index("s") * NC + lax.axis_index("c")
    base = wid * BPW

    for t in range(3):
        for j in range(4):
            pltpu.sync_copy(
                (users_hbm, pos_hbm, neg_hbm)[t].at[pl.ds(base + j * CHUNK, CHUNK)],
                raw.at[t * 4 + j])

    def shift(k, carry):
        sl = pl.ds(k * LANES, LANES)
        for c in range(12):
            gidx[c, sl] = lax.shift_right_logical(raw[c, sl], 1)
        return carry

    lax.fori_loop(0, CHUNK // LANES, shift, 0)

    lanes = lax.iota(jnp.int32, LANES)

    dnums = lax.GatherDimensionNumbers(
        offset_dims=(), collapsed_slice_dims=(0,), start_index_map=(0,))

    def permute(v, idx):
        return lax.gather(v, idx[:, None], dnums, (1,),
                          mode=lax.GatherScatterMode.PROMISE_IN_BOUNDS)

    perms = {h: lanes ^ h for h in (1, 2, 4, 8)}
    masks = {h: (lanes & h) != 0 for h in (1, 2, 4, 8)}

    def combine(a, b, h):
        return (jnp.where(masks[h], permute(b, perms[h]), a)
                + jnp.where(masks[h], b, permute(a, perms[h])))

    def tree(vs):
        h = 1
        while len(vs) > 1:
            vs = [combine(vs[i], vs[i + 1], h) for i in range(0, len(vs), 2)]
            h *= 2
        return vs[0]

    for half in range(2):
        hbase = half * HALF
        copies = []
        for j in range(NCHUNK):
            dst = pl.ds(j * CHUNK, CHUNK)
            copies.append(pltpu.async_copy(
                table2_hbm.at[gidx.at[0 * 4 + 2 * half + j]], urows.at[dst], sem))
            copies.append(pltpu.async_copy(
                item2_hbm.at[gidx.at[1 * 4 + 2 * half + j]], prows.at[dst], sem))
            copies.append(pltpu.async_copy(
                item2_hbm.at[gidx.at[2 * 4 + 2 * half + j]], nrows.at[dst], sem))
        for c in copies:
            c.wait()

        def group(g, carry):
            c_u = (2 * half) + g // (CHUNK // LANES)
            off = (g % (CHUNK // LANES)) * LANES
            uv = raw[0 * 4 + c_u, pl.ds(off, LANES)]
            pv = raw[1 * 4 + c_u, pl.ds(off, LANES)]
            nv = raw[2 * 4 + c_u, pl.ds(off, LANES)]
            sp, sn = [], []
            for j in range(LANES):
                row = g * LANES + j
                bu = (uv[j] & 1) * EMB
                bp = (pv[j] & 1) * EMB
                bn = (nv[j] & 1) * EMB
                pp = jnp.zeros((LANES,), jnp.float32)
                nn = jnp.zeros((LANES,), jnp.float32)
                for k in range(EMB // LANES):
                    u = urows[row, pl.ds(bu + k * LANES, LANES)]
                    pp = pp + u * prows[row, pl.ds(bp + k * LANES, LANES)]
                    nn = nn + u * nrows[row, pl.ds(bn + k * LANES, LANES)]
                sp.append(pp)
                sn.append(nn)
            sl = pl.ds(hbase + g * LANES, LANES)
            outp[sl] = tree(sp)
            outn[sl] = tree(sn)
            return carry

        lax.fori_loop(0, HALF // LANES, group, 0)

    pltpu.sync_copy(outp, pos_out_hbm.at[pl.ds(base, BPW)])
    pltpu.sync_copy(outn, neg_out_hbm.at[pl.ds(base, BPW)])


@functools.partial(
    pl.kernel,
    out_type=(
        jax.ShapeDtypeStruct((BATCH,), jnp.float32),
        jax.ShapeDtypeStruct((BATCH,), jnp.float32),
    ),
    mesh=plsc.VectorSubcoreMesh(core_axis_name="c", subcore_axis_name="s"),
    scratch_types=[
        pltpu.VMEM((12, CHUNK), jnp.int32),
        pltpu.VMEM((12, CHUNK), jnp.int32),
        pltpu.VMEM((HALF, ROWW), jnp.float32),
        pltpu.VMEM((HALF, ROWW), jnp.float32),
        pltpu.VMEM((HALF, ROWW), jnp.float32),
        pltpu.VMEM((BPW,), jnp.float32),
        pltpu.VMEM((BPW,), jnp.float32),
        pltpu.SemaphoreType.DMA,
    ],
)
def _scores_sc(users_hbm, pos_hbm, neg_hbm, item2_hbm, table2_hbm,
               pos_out_hbm, neg_out_hbm, *scratch):
    _sc_body(users_hbm, pos_hbm, neg_hbm, item2_hbm, table2_hbm,
             pos_out_hbm, neg_out_hbm, *scratch)


@jax.jit
def kernel(users, pos_items, neg_items, all_item_embs, user_table):
    users = users.astype(jnp.int32)
    pos_items = pos_items.astype(jnp.int32)
    neg_items = neg_items.astype(jnp.int32)
    item2 = all_item_embs.reshape(all_item_embs.shape[0] // 2, 2 * EMB)
    table2 = user_table.reshape(user_table.shape[0] // 2, 2 * EMB)
    pos_scores, neg_scores = _scores_sc(
        users, pos_items, neg_items, item2, table2)
    return (pos_scores, neg_scores)


# speedup vs baseline: 1.3713x; 1.3713x over previous
import functools

import jax
import jax.numpy as jnp
from jax import lax
from jax.experimental import pallas as pl
from jax.experimental.pallas import tpu as pltpu
from jax.experimental.pallas import tpu_sc as plsc

BATCH = 16384
EMB = 64
ROWW = 128
NC = 2
NS = 16
NW = NC * NS
BPW = BATCH // NW
HALF = BPW // 2
CHUNK = 128
NCHUNK = HALF // CHUNK
LANES = 16
WAVE = 8
BLK = 8


def _sc_body(users_hbm, pos_hbm, neg_hbm, item2_hbm, table_hbm,
             pos_out_hbm, neg_out_hbm,
             raw, gidx, ublk, urows, prows, nrows, outp, outn, sem):
    wid = lax.axis_index("s") * NC + lax.axis_index("c")
    base = wid * BPW

    for t in range(3):
        for j in range(4):
            pltpu.sync_copy(
                (users_hbm, pos_hbm, neg_hbm)[t].at[pl.ds(base + j * CHUNK, CHUNK)],
                raw.at[t * 4 + j])

    def shift(k, carry):
        sl = pl.ds(k * LANES, LANES)
        for c in range(4, 12):
            gidx[c, sl] = lax.shift_right_logical(raw[c, sl], 1)
        return carry

    lax.fori_loop(0, CHUNK // LANES, shift, 0)

    def uwave(w, carry):
        cw = w // (CHUNK // WAVE)
        off = (w % (CHUNK // WAVE)) * WAVE
        uv = raw[cw, pl.ds(off, WAVE)]
        for j in range(WAVE):
            blk = pl.multiple_of((uv[j] >> 3) * BLK, BLK)
            pltpu.make_async_copy(
                table_hbm.at[pl.ds(blk, BLK), :], ublk.at[j], sem).start()
        for j in range(WAVE):
            pltpu.make_async_copy(
                table_hbm.at[pl.ds(0, BLK), :], ublk.at[j], sem).wait()
        for j in range(WAVE):
            row = w * WAVE + j
            s = uv[j] & 7
            for k in range(EMB // LANES):
                urows[row // 2, pl.ds((j % 2) * EMB + k * LANES, LANES)] = (
                    ublk[j, s, pl.ds(k * LANES, LANES)])
        return carry

    lax.fori_loop(0, BPW // WAVE, uwave, 0)

    lanes = lax.iota(jnp.int32, LANES)

    dnums = lax.GatherDimensionNumbers(
        offset_dims=(), collapsed_slice_dims=(0,), start_index_map=(0,))

    def permute(v, idx):
        return lax.gather(v, idx[:, None], dnums, (1,),
                          mode=lax.GatherScatterMode.PROMISE_IN_BOUNDS)

    perms = {h: lanes ^ h for h in (1, 2, 4, 8)}
    masks = {h: (lanes & h) != 0 for h in (1, 2, 4, 8)}

    def combine(a, b, h):
        return (jnp.where(masks[h], permute(b, perms[h]), a)
                + jnp.where(masks[h], b, permute(a, perms[h])))

    def tree(vs):
        h = 1
        while len(vs) > 1:
            vs = [combine(vs[i], vs[i + 1], h) for i in range(0, len(vs), 2)]
            h *= 2
        return vs[0]

    for half in range(2):
        hbase = half * HALF
        copies = []
        for j in range(NCHUNK):
            dst = pl.ds(j * CHUNK, CHUNK)
            copies.append(pltpu.async_copy(
                item2_hbm.at[gidx.at[4 + 2 * half + j]], prows.at[dst], sem))
            copies.append(pltpu.async_copy(
                item2_hbm.at[gidx.at[8 + 2 * half + j]], nrows.at[dst], sem))
        for c in copies:
            c.wait()

        def group(g, carry):
            c_i = (2 * half) + g // (CHUNK // LANES)
            off = (g % (CHUNK // LANES)) * LANES
            pv = raw[4 + c_i, pl.ds(off, LANES)]
            nv = raw[8 + c_i, pl.ds(off, LANES)]
            sp, sn = [], []
            for j in range(LANES):
                row = g * LANES + j
                bp = (pv[j] & 1) * EMB
                bn = (nv[j] & 1) * EMB
                pp = jnp.zeros((LANES,), jnp.float32)
                nn = jnp.zeros((LANES,), jnp.float32)
                for k in range(EMB // LANES):
                    u = urows[(hbase + row) // 2, pl.ds((j % 2) * EMB + k * LANES, LANES)]
                    pp = pp + u * prows[row, pl.ds(bp + k * LANES, LANES)]
                    nn = nn + u * nrows[row, pl.ds(bn + k * LANES, LANES)]
                sp.append(pp)
                sn.append(nn)
            sl = pl.ds(hbase + g * LANES, LANES)
            outp[sl] = tree(sp)
            outn[sl] = tree(sn)
            return carry

        lax.fori_loop(0, HALF // LANES, group, 0)

    pltpu.sync_copy(outp, pos_out_hbm.at[pl.ds(base, BPW)])
    pltpu.sync_copy(outn, neg_out_hbm.at[pl.ds(base, BPW)])


@functools.partial(
    pl.kernel,
    out_type=(
        jax.ShapeDtypeStruct((BATCH,), jnp.float32),
        jax.ShapeDtypeStruct((BATCH,), jnp.float32),
    ),
    mesh=plsc.VectorSubcoreMesh(core_axis_name="c", subcore_axis_name="s"),
    scratch_types=[
        pltpu.VMEM((12, CHUNK), jnp.int32),
        pltpu.VMEM((12, CHUNK), jnp.int32),
        pltpu.VMEM((WAVE, BLK, EMB), jnp.float32),
        pltpu.VMEM((BPW // 2, 2 * EMB), jnp.float32),
        pltpu.VMEM((HALF, ROWW), jnp.float32),
        pltpu.VMEM((HALF, ROWW), jnp.float32),
        pltpu.VMEM((BPW,), jnp.float32),
        pltpu.VMEM((BPW,), jnp.float32),
        pltpu.SemaphoreType.DMA,
    ],
)
def _scores_sc(users_hbm, pos_hbm, neg_hbm, item2_hbm, table_hbm,
               pos_out_hbm, neg_out_hbm, *scratch):
    _sc_body(users_hbm, pos_hbm, neg_hbm, item2_hbm, table_hbm,
             pos_out_hbm, neg_out_hbm, *scratch)


@jax.jit
def kernel(users, pos_items, neg_items, all_item_embs, user_table):
    users = users.astype(jnp.int32)
    pos_items = pos_items.astype(jnp.int32)
    neg_items = neg_items.astype(jnp.int32)
    item2 = all_item_embs.reshape(all_item_embs.shape[0] // 2, 2 * EMB)
    pos_scores, neg_scores = _scores_sc(
        users, pos_items, neg_items, item2, user_table)
    return (pos_scores, neg_scores)


# speedup vs baseline: 2.1215x; 1.5471x over previous
import functools

import jax
import jax.numpy as jnp
from jax import lax
from jax.experimental import pallas as pl
from jax.experimental.pallas import tpu as pltpu
from jax.experimental.pallas import tpu_sc as plsc

BATCH = 16384
EMB = 64
ROWW = 128
NC = 2
NS = 16
NW = NC * NS
BPW = BATCH // NW
HALF = BPW // 2
CHUNK = 128
NCHUNK = HALF // CHUNK
LANES = 16
WAVE = 8
BLK = 8


def _sc_body(users_hbm, pos_hbm, neg_hbm, item2_hbm, table_hbm,
             pos_out_hbm, neg_out_hbm,
             raw, gidx, ublkA, ublkB, ublkC, urows, prows, nrows, outp, outn,
             sem_idx, sem_usrA, sem_usrB, sem_usrC, sem_itm):
    wid = lax.axis_index("s") * NC + lax.axis_index("c")
    base = wid * BPW

    idx_copies = []
    for t in range(3):
        for j in range(4):
            idx_copies.append(pltpu.make_async_copy(
                (users_hbm, pos_hbm, neg_hbm)[t].at[pl.ds(base + j * CHUNK, CHUNK)],
                raw.at[t * 4 + j], sem_idx))
    for c in idx_copies:
        c.start()
    for c in idx_copies:
        c.wait()

    def shift(k, carry):
        sl = pl.ds(k * LANES, LANES)
        for c in range(4, 12):
            gidx[c, sl] = lax.shift_right_logical(raw[c, sl], 1)
        return carry

    lax.fori_loop(0, CHUNK // LANES, shift, 0)

    def fire_items(half):
        cs = []
        for j in range(NCHUNK):
            dst = pl.ds(j * CHUNK, CHUNK)
            cs.append(pltpu.async_copy(
                item2_hbm.at[gidx.at[4 + 2 * half + j]], prows.at[dst], sem_itm))
            cs.append(pltpu.async_copy(
                item2_hbm.at[gidx.at[8 + 2 * half + j]], nrows.at[dst], sem_itm))
        return cs

    items0 = fire_items(0)

    def issue(w, buf, sem):
        cw = w // (CHUNK // WAVE)
        off = (w % (CHUNK // WAVE)) * WAVE
        uv = raw[cw, pl.ds(off, WAVE)]
        for j in range(WAVE):
            pltpu.make_async_copy(
                table_hbm.at[uv[j] >> 3], buf.at[j], sem).start()
        return uv

    def drain(buf, sem):
        pltpu.make_async_copy(table_hbm.at[pl.ds(0, WAVE)], buf, sem).wait()

    def extract(w, buf, uv):
        for j in range(WAVE):
            row = w * WAVE + j
            s = uv[j] & 7
            for k in range(EMB // LANES):
                urows[row // 2, pl.ds((j % 2) * EMB + k * LANES, LANES)] = (
                    buf[j, s, pl.ds(k * LANES, LANES)])

    nw = BPW // WAVE
    uvA = issue(0, ublkA, sem_usrA)
    uvB = issue(1, ublkB, sem_usrB)

    def uwave3(i, uvs):
        uvA, uvB = uvs
        uvC = issue(3 * i + 2, ublkC, sem_usrC)
        drain(ublkA, sem_usrA)
        extract(3 * i, ublkA, uvA)
        uvA = issue(3 * i + 3, ublkA, sem_usrA)
        drain(ublkB, sem_usrB)
        extract(3 * i + 1, ublkB, uvB)
        uvB = issue(3 * i + 4, ublkB, sem_usrB)
        drain(ublkC, sem_usrC)
        extract(3 * i + 2, ublkC, uvC)
        return (uvA, uvB)

    uvA, uvB = lax.fori_loop(0, (nw - 4) // 3, uwave3, (uvA, uvB))
    uvC = issue(nw - 2, ublkC, sem_usrC)
    drain(ublkA, sem_usrA)
    extract(nw - 4, ublkA, uvA)
    uvA = issue(nw - 1, ublkA, sem_usrA)
    drain(ublkB, sem_usrB)
    extract(nw - 3, ublkB, uvB)
    drain(ublkC, sem_usrC)
    extract(nw - 2, ublkC, uvC)
    drain(ublkA, sem_usrA)
    extract(nw - 1, ublkA, uvA)

    lanes = lax.iota(jnp.int32, LANES)

    dnums = lax.GatherDimensionNumbers(
        offset_dims=(), collapsed_slice_dims=(0,), start_index_map=(0,))

    def permute(v, idx):
        return lax.gather(v, idx[:, None], dnums, (1,),
                          mode=lax.GatherScatterMode.PROMISE_IN_BOUNDS)

    perms = {h: lanes ^ h for h in (1, 2, 4, 8)}
    masks = {h: (lanes & h) != 0 for h in (1, 2, 4, 8)}

    def combine(a, b, h):
        return (jnp.where(masks[h], permute(b, perms[h]), a)
                + jnp.where(masks[h], b, permute(a, perms[h])))

    def tree(vs):
        h = 1
        while len(vs) > 1:
            vs = [combine(vs[i], vs[i + 1], h) for i in range(0, len(vs), 2)]
            h *= 2
        return vs[0]

    for half in range(2):
        hbase = half * HALF
        for c in (items0 if half == 0 else items1):
            c.wait()

        def group(g, carry):
            c_i = (2 * half) + g // (CHUNK // LANES)
            off = (g % (CHUNK // LANES)) * LANES
            pv = raw[4 + c_i, pl.ds(off, LANES)]
            nv = raw[8 + c_i, pl.ds(off, LANES)]
            sp, sn = [], []
            for j in range(LANES):
                row = g * LANES + j
                bp = (pv[j] & 1) * EMB
                bn = (nv[j] & 1) * EMB
                pp = jnp.zeros((LANES,), jnp.float32)
                nn = jnp.zeros((LANES,), jnp.float32)
                for k in range(EMB // LANES):
                    u = urows[(hbase + row) // 2, pl.ds((j % 2) * EMB + k * LANES, LANES)]
                    pp = pp + u * prows[row, pl.ds(bp + k * LANES, LANES)]
                    nn = nn + u * nrows[row, pl.ds(bn + k * LANES, LANES)]
                sp.append(pp)
                sn.append(nn)
            sl = pl.ds(hbase + g * LANES, LANES)
            outp[sl] = tree(sp)
            outn[sl] = tree(sn)
            return carry

        lax.fori_loop(0, HALF // LANES, group, 0)
        if half == 0:
            items1 = fire_items(1)

    pltpu.sync_copy(outp, pos_out_hbm.at[pl.ds(base, BPW)])
    pltpu.sync_copy(outn, neg_out_hbm.at[pl.ds(base, BPW)])


@functools.partial(
    pl.kernel,
    out_type=(
        jax.ShapeDtypeStruct((BATCH,), jnp.float32),
        jax.ShapeDtypeStruct((BATCH,), jnp.float32),
    ),
    mesh=plsc.VectorSubcoreMesh(core_axis_name="c", subcore_axis_name="s"),
    scratch_types=[
        pltpu.VMEM((12, CHUNK), jnp.int32),
        pltpu.VMEM((12, CHUNK), jnp.int32),
        pltpu.VMEM((WAVE, BLK, EMB), jnp.float32),
        pltpu.VMEM((WAVE, BLK, EMB), jnp.float32),
        pltpu.VMEM((WAVE, BLK, EMB), jnp.float32),
        pltpu.VMEM((BPW // 2, 2 * EMB), jnp.float32),
        pltpu.VMEM((HALF, ROWW), jnp.float32),
        pltpu.VMEM((HALF, ROWW), jnp.float32),
        pltpu.VMEM((BPW,), jnp.float32),
        pltpu.VMEM((BPW,), jnp.float32),
        pltpu.SemaphoreType.DMA,
        pltpu.SemaphoreType.DMA,
        pltpu.SemaphoreType.DMA,
        pltpu.SemaphoreType.DMA,
        pltpu.SemaphoreType.DMA,
    ],
)
def _scores_sc(users_hbm, pos_hbm, neg_hbm, item2_hbm, table_hbm,
               pos_out_hbm, neg_out_hbm, *scratch):
    _sc_body(users_hbm, pos_hbm, neg_hbm, item2_hbm, table_hbm,
             pos_out_hbm, neg_out_hbm, *scratch)


@jax.jit
def kernel(users, pos_items, neg_items, all_item_embs, user_table):
    users = users.astype(jnp.int32)
    pos_items = pos_items.astype(jnp.int32)
    neg_items = neg_items.astype(jnp.int32)
    item2 = all_item_embs.reshape(all_item_embs.shape[0] // 2, 2 * EMB)
    table3 = user_table.reshape(user_table.shape[0] // BLK, BLK, EMB)
    pos_scores, neg_scores = _scores_sc(
        users, pos_items, neg_items, item2, table3)
    return (pos_scores, neg_scores)
